# Initial kernel scaffold; baseline (speedup 1.0000x reference)
import jax
import jax.numpy as jnp
from jax.experimental import pallas as pl


def kernel(node_feats, edge_feats, edge_index, Weu1, beu1, Weu2, beu2, Wn1, bn1, We1, be1, We2, be2, Wn2a, bn2a, Wn2b, bn2b):
    raise NotImplementedError("write your pallas kernel here")


# speedup vs baseline: 1.6714x; 1.6714x over previous
import functools

import jax
import jax.numpy as jnp
from jax import lax
from jax.experimental import pallas as pl
from jax.experimental.pallas import tpu as pltpu
from jax.experimental.pallas import tpu_sc as plsc

NC = 2
NS = 16
NW = NC * NS
LN2 = 0.6931471805599453


def _ssp(x):
    return jnp.maximum(x, 0.0) + jnp.log(1.0 + jnp.exp(-jnp.abs(x))) - LN2


def _sc_gather(node_feats, idx_flat, chunk=80):
    R = idx_flat.shape[0]
    D = node_feats.shape[1]
    per_w = R // NW
    n_chunks = per_w // chunk
    mesh = plsc.VectorSubcoreMesh(
        core_axis_name="c", subcore_axis_name="s",
        num_cores=NC, num_subcores=NS)

    @functools.partial(
        pl.kernel,
        out_type=jax.ShapeDtypeStruct((R, D), jnp.float32),
        mesh=mesh,
        compiler_params=pltpu.CompilerParams(use_tc_tiling_on_sc=False),
        scratch_types=[
            pltpu.VMEM((chunk,), jnp.int32),
            pltpu.VMEM((chunk, D), jnp.float32),
            pltpu.SemaphoreType.DMA,
        ],
    )
    def k(nf_hbm, idx_hbm, out_hbm, idx_v, rows_v, sem):
        wid = lax.axis_index("s") * NC + lax.axis_index("c")
        base = wid * per_w

        def body(j, carry):
            off = pl.multiple_of(base + j * chunk, 8)
            pltpu.sync_copy(idx_hbm.at[pl.ds(off, chunk)], idx_v)
            pltpu.async_copy(nf_hbm.at[idx_v], rows_v, sem).wait()
            pltpu.sync_copy(rows_v, out_hbm.at[pl.ds(off, chunk)])
            return carry

        lax.fori_loop(0, n_chunks, body, 0)

    return k(node_feats, idx_flat)


def _tc_edge_mlp(h3, ef, W1s, W1d, W1e, b1, W2, b2, We1, be1, We2, be2,
                 blk=1600):
    E, R = ef.shape
    D = h3.shape[2]
    H = D // 2
    grid = (E // blk,)

    def body(hs_ref, hd_ref, ef_ref, W1s_r, W1d_r, W1e_r, b1_r, W2_r, b2_r,
             We1_r, be1_r, We2_r, be2_r, en_ref, he_ref):
        hs = hs_ref[0]
        hd = hd_ref[0]
        u = (jnp.dot(hs, W1s_r[...], preferred_element_type=jnp.float32)
             + jnp.dot(hd, W1d_r[...], preferred_element_type=jnp.float32)
             + jnp.dot(ef_ref[...], W1e_r[...], preferred_element_type=jnp.float32)
             + b1_r[...])
        u = _ssp(u)
        en = jnp.dot(u, W2_r[...], preferred_element_type=jnp.float32) + b2_r[...]
        en_ref[...] = en
        t = _ssp(jnp.dot(en, We1_r[...], preferred_element_type=jnp.float32)
                 + be1_r[...])
        he = jnp.dot(t, We2_r[...], preferred_element_type=jnp.float32) + be2_r[...]
        he_ref[0] = he[:, :H]
        he_ref[1] = he[:, H:]

    full = lambda shape: pl.BlockSpec(shape, lambda i: (0,) * len(shape))
    return pl.pallas_call(
        body,
        grid=grid,
        in_specs=[
            pl.BlockSpec((1, blk, D), lambda i: (0, i, 0)),
            pl.BlockSpec((1, blk, D), lambda i: (1, i, 0)),
            pl.BlockSpec((blk, R), lambda i: (i, 0)),
            full(W1s.shape), full(W1d.shape), full(W1e.shape), full(b1.shape),
            full(W2.shape), full(b2.shape), full(We1.shape), full(be1.shape),
            full(We2.shape), full(be2.shape),
        ],
        out_specs=[
            pl.BlockSpec((blk, D), lambda i: (i, 0)),
            pl.BlockSpec((2, blk, H), lambda i: (0, i, 0)),
        ],
        out_shape=[
            jax.ShapeDtypeStruct((E, D), jnp.float32),
            jax.ShapeDtypeStruct((2, E, H), jnp.float32),
        ],
    )(h3, h3, ef, W1s, W1d, W1e, b1, W2, b2, We1, be1, We2, be2)


def _sc_scatter(he2, dst, zeros, chunk=80):
    _, E, H = he2.shape
    N = zeros.shape[0]
    per_s = E // NS
    n_chunks = per_s // chunk
    rows_per_s = N // NS
    mesh = plsc.VectorSubcoreMesh(
        core_axis_name="c", subcore_axis_name="s",
        num_cores=NC, num_subcores=NS)

    @functools.partial(
        pl.kernel,
        out_type=jax.ShapeDtypeStruct((NC, N, H), jnp.float32),
        mesh=mesh,
        compiler_params=pltpu.CompilerParams(use_tc_tiling_on_sc=False),
        scratch_types=[
            pltpu.VMEM((chunk,), jnp.int32),
            pltpu.VMEM((chunk, H), jnp.float32),
            pltpu.VMEM_SHARED((N, H), jnp.float32),
            pltpu.SemaphoreType.DMA,
        ],
    )
    def k(he_hbm, dst_hbm, z_hbm, out_hbm, idx_v, rows_v, acc, sem):
        c = lax.axis_index("c")
        s = lax.axis_index("s")
        r0 = pl.multiple_of(s * rows_per_s, 8)
        pltpu.sync_copy(z_hbm.at[pl.ds(r0, rows_per_s)],
                        acc.at[pl.ds(r0, rows_per_s)])
        plsc.subcore_barrier()

        def body(j, carry):
            off = pl.multiple_of(s * per_s + j * chunk, 8)
            pltpu.sync_copy(dst_hbm.at[pl.ds(off, chunk)], idx_v)
            pltpu.sync_copy(he_hbm.at[c].at[pl.ds(off, chunk)], rows_v)
            pltpu.async_copy(rows_v, acc.at[idx_v], sem, add=True).wait()
            return carry

        lax.fori_loop(0, n_chunks, body, 0)
        plsc.subcore_barrier()
        pltpu.sync_copy(acc.at[pl.ds(r0, rows_per_s)],
                        out_hbm.at[c].at[pl.ds(r0, rows_per_s)])

    return k(he2, dst, zeros)


def _tc_node_mlp(node_feats, agg2, Wa, ba, Wb, bb, blk=2000):
    N, D = node_feats.shape
    H = D // 2
    grid = (N // blk,)

    def body(nf_ref, g0_ref, g1_ref, Wa_r, ba_r, Wb_r, bb_r, out_ref):
        a = jnp.concatenate([g0_ref[0], g1_ref[0]], axis=1)
        t = _ssp(jnp.dot(a, Wa_r[...], preferred_element_type=jnp.float32)
                 + ba_r[...])
        out_ref[...] = (nf_ref[...] + bb_r[...]
                        + jnp.dot(t, Wb_r[...], preferred_element_type=jnp.float32))

    full = lambda shape: pl.BlockSpec(shape, lambda i: (0,) * len(shape))
    return pl.pallas_call(
        body,
        grid=grid,
        in_specs=[
            pl.BlockSpec((blk, D), lambda i: (i, 0)),
            pl.BlockSpec((1, blk, H), lambda i: (0, i, 0)),
            pl.BlockSpec((1, blk, H), lambda i: (1, i, 0)),
            full(Wa.shape), full(ba.shape), full(Wb.shape), full(bb.shape),
        ],
        out_specs=pl.BlockSpec((blk, D), lambda i: (i, 0)),
        out_shape=jax.ShapeDtypeStruct((N, D), jnp.float32),
    )(node_feats, agg2, agg2, Wa, ba, Wb, bb)


def kernel(node_feats, edge_feats, edge_index, Weu1, beu1, Weu2, beu2,
           Wn1, bn1, We1, be1, We2, be2, Wn2a, bn2a, Wn2b, bn2b):
    N, D = node_feats.shape
    E = edge_feats.shape[0]
    H = D // 2

    idx_flat = edge_index.reshape(2 * E)
    h_cat = _sc_gather(node_feats, idx_flat)
    h3 = h_cat.reshape(2, E, D)

    W1s = Weu1[:D]
    W1d = Weu1[D:2 * D]
    W1e = Weu1[2 * D:]
    edge_new, he2 = _tc_edge_mlp(
        h3, edge_feats, W1s, W1d, W1e, beu1.reshape(1, -1),
        Weu2, beu2.reshape(1, -1), We1, be1.reshape(1, -1),
        We2, be2.reshape(1, -1))

    zeros = jnp.zeros((N, H), jnp.float32)
    agg2 = _sc_scatter(he2, edge_index[1], zeros)

    node_out = _tc_node_mlp(node_feats, agg2,
                            Wn2a, bn2a.reshape(1, -1),
                            Wn2b, bn2b.reshape(1, -1))
    return (node_out, edge_new)


# speedup vs baseline: 2.9394x; 1.7587x over previous
import functools

import jax
import jax.numpy as jnp
from jax import lax
from jax.experimental import pallas as pl
from jax.experimental.pallas import tpu as pltpu
from jax.experimental.pallas import tpu_sc as plsc

NC = 2
NS = 16
NW = NC * NS
LN2 = 0.6931471805599453


def _ssp(x):
    return jnp.maximum(x, 0.0) + jnp.log(1.0 + jnp.exp(-jnp.abs(x))) - LN2


def _full(shape):
    return pl.BlockSpec(shape, lambda i: (0,) * len(shape))


def _mesh():
    return plsc.VectorSubcoreMesh(
        core_axis_name="c", subcore_axis_name="s",
        num_cores=NC, num_subcores=NS)


def _tc_project(nf, W1s, W1d, b1, blk=2000):
    N, D = nf.shape
    K = W1s.shape[1]

    def body(nf_ref, Ws_r, Wd_r, b1_r, ps_ref, pd_ref):
        x = nf_ref[...]
        ps_ref[...] = jnp.dot(x, Ws_r[...], preferred_element_type=jnp.float32)
        pd_ref[...] = (jnp.dot(x, Wd_r[...], preferred_element_type=jnp.float32)
                       + b1_r[...])

    return pl.pallas_call(
        body,
        grid=(N // blk,),
        in_specs=[pl.BlockSpec((blk, D), lambda i: (i, 0)),
                  _full(W1s.shape), _full(W1d.shape), _full(b1.shape)],
        out_specs=[pl.BlockSpec((blk, K), lambda i: (i, 0)),
                   pl.BlockSpec((blk, K), lambda i: (i, 0))],
        out_shape=[jax.ShapeDtypeStruct((N, K), jnp.float32),
                   jax.ShapeDtypeStruct((N, K), jnp.float32)],
    )(nf, W1s, W1d, b1)


def _sc_gather(Ps, Pd, src, dst, CH=128):
    E = src.shape[0]
    K = Ps.shape[1]
    total = E // CH
    base_n = total // NW
    extra = total - base_n * NW

    @functools.partial(
        pl.kernel,
        out_type=jax.ShapeDtypeStruct((E, K), jnp.float32),
        mesh=_mesh(),
        compiler_params=pltpu.CompilerParams(use_tc_tiling_on_sc=False),
        scratch_types=[
            pltpu.VMEM((CH,), jnp.int32), pltpu.VMEM((CH,), jnp.int32),
            pltpu.VMEM((CH,), jnp.int32), pltpu.VMEM((CH,), jnp.int32),
            pltpu.VMEM((CH, K), jnp.float32), pltpu.VMEM((CH, K), jnp.float32),
            pltpu.VMEM((CH, K), jnp.float32), pltpu.VMEM((CH, K), jnp.float32),
            pltpu.SemaphoreType.DMA, pltpu.SemaphoreType.DMA,
            pltpu.SemaphoreType.DMA, pltpu.SemaphoreType.DMA,
        ],
    )
    def k(ps_hbm, pd_hbm, src_hbm, dst_hbm, out_hbm,
          is0, is1, id0, id1, ra0, ra1, rb0, rb1, sg0, sg1, sw0, sw1):
        wid = lax.axis_index("s") * NC + lax.axis_index("c")
        my_n = jnp.where(wid < extra, base_n + 1, base_n)
        isv, idv = (is0, is1), (id0, id1)
        ra, rb = (ra0, ra1), (rb0, rb1)
        sg, sw = (sg0, sg1), (sw0, sw1)

        def start_stage(j, b):
            @pl.when(j < my_n)
            def _():
                @pl.when(j >= 2)
                def _():
                    pltpu.make_async_copy(ra[b], out_hbm.at[pl.ds(0, CH)],
                                          sw[b]).wait()
                off = pl.multiple_of((wid + j * NW) * CH, CH)
                pltpu.sync_copy(src_hbm.at[pl.ds(off, CH)], isv[b])
                pltpu.sync_copy(dst_hbm.at[pl.ds(off, CH)], idv[b])
                pltpu.async_copy(ps_hbm.at[isv[b]], ra[b], sg[b])
                pltpu.async_copy(pd_hbm.at[idv[b]], rb[b], sg[b])

        def compute_stage(j, b):
            @pl.when((j >= 0) & (j < my_n))
            def _():
                pltpu.make_async_copy(ps_hbm.at[isv[b]], ra[b], sg[b]).wait()
                pltpu.make_async_copy(pd_hbm.at[idv[b]], rb[b], sg[b]).wait()

                def addrows(r8, carry):
                    for r0 in range(8):
                        for t in range(K // 16):
                            sl = pl.ds(t * 16, 16)
                            plsc.addupdate(ra[b].at[r8 * 8 + r0, sl],
                                           rb[b][r8 * 8 + r0, sl])
                    return carry

                lax.fori_loop(0, CH // 8, addrows, 0)
                off = pl.multiple_of((wid + j * NW) * CH, CH)
                pltpu.async_copy(ra[b], out_hbm.at[pl.ds(off, CH)], sw[b])

        def body(jj, carry):
            for b in (0, 1):
                j = 2 * jj + b
                start_stage(j, b)
                compute_stage(j - 1, 1 - b)
            return carry

        lax.fori_loop(0, (base_n + 1) // 2 + 1, body, 0)
        pltpu.make_async_copy(ra0, out_hbm.at[pl.ds(0, CH)], sw0).wait()
        pltpu.make_async_copy(ra1, out_hbm.at[pl.ds(0, CH)], sw1).wait()

    return k(Ps, Pd, src, dst)


def _tc_edge_mlp(u_pre, ef, W1e, W2, b2, We1, be1, We2, be2, blk=1600):
    E, R = ef.shape
    K = u_pre.shape[1]
    D = W2.shape[1]

    def body(up_ref, ef_ref, W1e_r, W2_r, b2_r, We1_r, be1_r, We2_r, be2_r,
             en_ref, he_ref):
        u = _ssp(up_ref[...]
                 + jnp.dot(ef_ref[...], W1e_r[...],
                           preferred_element_type=jnp.float32))
        en = jnp.dot(u, W2_r[...], preferred_element_type=jnp.float32) + b2_r[...]
        en_ref[...] = en
        t = _ssp(jnp.dot(en, We1_r[...], preferred_element_type=jnp.float32)
                 + be1_r[...])
        he_ref[...] = (jnp.dot(t, We2_r[...], preferred_element_type=jnp.float32)
                       + be2_r[...])

    return pl.pallas_call(
        body,
        grid=(E // blk,),
        in_specs=[
            pl.BlockSpec((blk, K), lambda i: (i, 0)),
            pl.BlockSpec((blk, R), lambda i: (i, 0)),
            _full(W1e.shape), _full(W2.shape), _full(b2.shape),
            _full(We1.shape), _full(be1.shape), _full(We2.shape),
            _full(be2.shape),
        ],
        out_specs=[
            pl.BlockSpec((blk, R), lambda i: (i, 0)),
            pl.BlockSpec((blk, D), lambda i: (i, 0)),
        ],
        out_shape=[
            jax.ShapeDtypeStruct((E, R), jnp.float32),
            jax.ShapeDtypeStruct((E, D), jnp.float32),
        ],
    )(u_pre, ef, W1e, W2, b2, We1, be1, We2, be2)


def _tc_remap(dst2, HALF):
    RN, RL = dst2.shape

    def body(d_ref, o_ref):
        v = d_ref[...]
        pad = HALF + (v & 15)
        o_ref[0] = jnp.where(v < HALF, v, pad)
        v1 = v - HALF
        o_ref[1] = jnp.where(v1 >= 0, v1, pad)

    return pl.pallas_call(
        body,
        in_specs=[pl.BlockSpec((RN, RL), lambda: (0, 0))],
        out_specs=pl.BlockSpec((2, RN, RL), lambda: (0, 0, 0)),
        out_shape=jax.ShapeDtypeStruct((2, RN, RL), jnp.int32),
    )(dst2)


def _sc_scatter(he, dl, zeros, N):
    E, D = he.shape
    CH = dl.shape[2]
    HALF = N // NC
    ACC = zeros.shape[0]
    total = E // CH
    base_n = total // NS
    extra = total - base_n * NS
    DR = 1568

    @functools.partial(
        pl.kernel,
        out_type=jax.ShapeDtypeStruct((N, D), jnp.float32),
        mesh=_mesh(),
        compiler_params=pltpu.CompilerParams(use_tc_tiling_on_sc=False),
        scratch_types=[
            pltpu.VMEM((CH,), jnp.int32), pltpu.VMEM((CH,), jnp.int32),
            pltpu.VMEM((CH, D), jnp.float32), pltpu.VMEM((CH, D), jnp.float32),
            pltpu.VMEM_SHARED((ACC, D), jnp.float32),
            pltpu.SemaphoreType.DMA, pltpu.SemaphoreType.DMA,
            pltpu.SemaphoreType.DMA,
        ],
    )
    def k(he_hbm, dl_hbm, z_hbm, out_hbm,
          iv0, iv1, rv0, rv1, acc, si0, si1, ss):
        c = lax.axis_index("c")
        s = lax.axis_index("s")
        node0 = c * HALF
        my_n = jnp.where(s < extra, base_n + 1, base_n)
        iv, rv = (iv0, iv1), (rv0, rv1)
        si = (si0, si1)
        z0 = pl.multiple_of(s * DR, 8)

        @pl.when(s < NS - 1)
        def _():
            pltpu.sync_copy(z_hbm.at[pl.ds(z0, DR)], acc.at[pl.ds(z0, DR)])

        @pl.when(s == NS - 1)
        def _():
            zl = pl.multiple_of((NS - 1) * DR, 8)
            n = ACC - (NS - 1) * DR
            pltpu.sync_copy(z_hbm.at[pl.ds(zl, n)], acc.at[pl.ds(zl, n)])

        plsc.subcore_barrier()

        def start_stage(j, b):
            @pl.when(j < my_n)
            def _():
                row = s + j * NS
                off = pl.multiple_of(row * CH, CH)
                pltpu.async_copy(dl_hbm.at[c, row], iv[b], si[b])
                pltpu.async_copy(he_hbm.at[pl.ds(off, CH)], rv[b], si[b])

        def compute_stage(j, b):
            @pl.when((j >= 0) & (j < my_n))
            def _():
                row = s + j * NS
                off = pl.multiple_of(row * CH, CH)
                pltpu.make_async_copy(dl_hbm.at[c, row], iv[b], si[b]).wait()
                pltpu.make_async_copy(he_hbm.at[pl.ds(off, CH)], rv[b],
                                      si[b]).wait()
                pltpu.async_copy(rv[b], acc.at[iv[b]], ss, add=True).wait()

        def body(jj, carry):
            for b in (0, 1):
                j = 2 * jj + b
                start_stage(j, b)
                compute_stage(j - 1, 1 - b)
            return carry

        lax.fori_loop(0, (base_n + 1) // 2 + 1, body, 0)
        plsc.subcore_barrier()

        @pl.when(s < NS - 1)
        def _():
            pltpu.sync_copy(acc.at[pl.ds(z0, DR)],
                            out_hbm.at[pl.ds(node0 + z0, DR)])

        @pl.when(s == NS - 1)
        def _():
            zl = pl.multiple_of((NS - 1) * DR, 8)
            n = HALF - (NS - 1) * DR
            pltpu.sync_copy(acc.at[pl.ds(zl, n)],
                            out_hbm.at[pl.ds(node0 + zl, n)])

    return k(he, dl, zeros)


def _tc_node_mlp(node_feats, agg, Wa, ba, Wb, bb, blk=2000):
    N, D = node_feats.shape

    def body(nf_ref, g_ref, Wa_r, ba_r, Wb_r, bb_r, out_ref):
        t = _ssp(jnp.dot(g_ref[...], Wa_r[...],
                         preferred_element_type=jnp.float32) + ba_r[...])
        out_ref[...] = (nf_ref[...] + bb_r[...]
                        + jnp.dot(t, Wb_r[...],
                                  preferred_element_type=jnp.float32))

    return pl.pallas_call(
        body,
        grid=(N // blk,),
        in_specs=[
            pl.BlockSpec((blk, D), lambda i: (i, 0)),
            pl.BlockSpec((blk, D), lambda i: (i, 0)),
            _full(Wa.shape), _full(ba.shape), _full(Wb.shape), _full(bb.shape),
        ],
        out_specs=pl.BlockSpec((blk, D), lambda i: (i, 0)),
        out_shape=jax.ShapeDtypeStruct((N, D), jnp.float32),
    )(node_feats, agg, Wa, ba, Wb, bb)


def kernel(node_feats, edge_feats, edge_index, Weu1, beu1, Weu2, beu2,
           Wn1, bn1, We1, be1, We2, be2, Wn2a, bn2a, Wn2b, bn2b):
    N, D = node_feats.shape
    E = edge_feats.shape[0]

    Ps, Pd = _tc_project(node_feats, Weu1[:D], Weu1[D:2 * D],
                         beu1.reshape(1, -1))
    src = edge_index[0]
    dst = edge_index[1]
    u_pre = _sc_gather(Ps, Pd, src, dst)

    edge_new, he = _tc_edge_mlp(
        u_pre, edge_feats, Weu1[2 * D:], Weu2, beu2.reshape(1, -1),
        We1, be1.reshape(1, -1), We2, be2.reshape(1, -1))

    dl = _tc_remap(dst.reshape(E // 128, 128), N // NC)
    dl = dl.reshape(2, E // 64, 64)
    zeros = jnp.zeros((N // NC + 16, D), jnp.float32)
    agg = _sc_scatter(he, dl, zeros, N)

    node_out = _tc_node_mlp(node_feats, agg,
                            Wn2a, bn2a.reshape(1, -1),
                            Wn2b, bn2b.reshape(1, -1))
    return (node_out, edge_new)
